# baseline (speedup 1.0000x reference)
import functools
import math

import jax
import jax.numpy as jnp
from jax import lax
from jax.experimental import pallas as pl
from jax.experimental.pallas import tpu as pltpu
from jax.experimental.pallas import tpu_sc as plsc

D_MODEL = 64
SCALE = math.sqrt(D_MODEL)
LANES = 16
CCHUNK = 128
NBUF = 2


@functools.lru_cache(maxsize=None)
def _build(n_i, n_j):
    info = plsc.get_sparse_core_info()
    nc, ns = info.num_cores, info.num_subcores
    nw = nc * ns
    n_it = n_i // CCHUNK
    nblocks = n_j * n_it
    bpw = nblocks // nw
    assert n_it * CCHUNK == n_i and bpw * nw == nblocks
    d_hi = D_MODEL // 8

    mesh = plsc.VectorSubcoreMesh(core_axis_name="c", subcore_axis_name="s")

    @functools.partial(
        pl.kernel,
        mesh=mesh,
        compiler_params=pltpu.CompilerParams(
            use_tc_tiling_on_sc=False, needs_layout_passes=False),
        out_type=jax.ShapeDtypeStruct((n_j, d_hi, n_it, 8, CCHUNK),
                                      jnp.float32),
        scratch_types=[
            pltpu.VMEM((bpw, CCHUNK), jnp.int32),
            pltpu.VMEM((NBUF, CCHUNK, D_MODEL), jnp.float32),
            pltpu.VMEM((NBUF, d_hi, 8, CCHUNK), jnp.float32),
            pltpu.SemaphoreType.DMA,
            pltpu.SemaphoreType.DMA,
            pltpu.SemaphoreType.DMA,
            pltpu.SemaphoreType.DMA,
        ],
    )
    def emb_kernel(x_hbm, table_hbm, out_hbm, idx_v, gbuf, tbuf,
                   gs0, gs1, os0, os1):
        gsems = (gs0, gs1)
        osems = (os0, os1)
        wid = lax.axis_index("s") * nc + lax.axis_index("c")
        block0 = wid * bpw

        pltpu.sync_copy(x_hbm.at[wid], idx_v)

        def gather(t, b):
            pltpu.async_copy(table_hbm.at[idx_v.at[t]], gbuf.at[b], gsems[b])

        def gwait(t, b):
            pltpu.make_async_copy(
                table_hbm.at[idx_v.at[t]], gbuf.at[b], gsems[b]).wait()

        def _out_slice(t):
            bid = block0 + t
            return out_hbm.at[bid // n_it, :, bid % n_it, :, :]

        def out_start(t, b):
            pltpu.async_copy(tbuf.at[b], _out_slice(t), osems[b])

        def owait(t, b):
            pltpu.make_async_copy(tbuf.at[b], _out_slice(t), osems[b]).wait()

        iota = lax.iota(jnp.int32, LANES)

        def transform(b):
            for k in range(CCHUNK // LANES):
                rowv = iota + (LANES * k)

                def _body(f, carry):
                    colv = jnp.broadcast_to(f, (LANES,))
                    v = plsc.load_gather(gbuf.at[b], [rowv, colv])
                    tbuf[b, f // 8, f % 8, pl.ds(LANES * k, LANES)] = (
                        v * SCALE)
                    return carry

                lax.fori_loop(0, D_MODEL, _body, 0, unroll=8)

        for b in range(NBUF):
            gather(b, b)
        for b in range(NBUF):
            gwait(b, b)
            transform(b)
            out_start(b, b)
            gather(b + NBUF, b)

        def block_pair(i, carry):
            for b in range(NBUF):
                t = i * NBUF + b
                gwait(t, b)
                owait(t - NBUF, b)
                transform(b)
                out_start(t, b)

                @pl.when(t + NBUF < bpw)
                def _():
                    gather(t + NBUF, b)
            return carry

        lax.fori_loop(1, bpw // NBUF, block_pair, 0)

        for b in range(NBUF):
            owait(bpw - NBUF + b, b)

    return emb_kernel, nw, n_it


def kernel(x, table):
    n_i, n_j = x.shape
    emb, nw, n_it = _build(n_i, n_j)
    x_t = x.T.reshape(nw, -1, CCHUNK)
    out5 = emb(x_t, table)
    return jnp.transpose(out5, (2, 4, 0, 1, 3)).reshape(n_i, n_j, D_MODEL)


# speedup vs baseline: 2.5485x; 2.5485x over previous
import functools
import math

import jax
import jax.numpy as jnp
from jax import lax
from jax.experimental import pallas as pl
from jax.experimental.pallas import tpu as pltpu
from jax.experimental.pallas import tpu_sc as plsc

D_MODEL = 64
SCALE = math.sqrt(D_MODEL)
LANES = 16
CCHUNK = 128
NBUF = 2


@functools.lru_cache(maxsize=None)
def _build(n_i, n_j):
    info = plsc.get_sparse_core_info()
    nc, ns = info.num_cores, info.num_subcores
    nw = nc * ns
    n_it = n_i // CCHUNK
    nblocks = n_j * n_it
    bpw = nblocks // nw
    assert n_it * CCHUNK == n_i and bpw * nw == nblocks
    d_hi = D_MODEL // 8

    mesh = plsc.VectorSubcoreMesh(core_axis_name="c", subcore_axis_name="s")

    @functools.partial(
        pl.kernel,
        mesh=mesh,
        compiler_params=pltpu.CompilerParams(
            use_tc_tiling_on_sc=False, needs_layout_passes=False),
        out_type=jax.ShapeDtypeStruct((n_j, d_hi, n_it, 8, CCHUNK),
                                      jnp.float32),
        scratch_types=[
            pltpu.VMEM((bpw, CCHUNK), jnp.int32),
            pltpu.VMEM((NBUF, CCHUNK, D_MODEL), jnp.float32),
            pltpu.VMEM((NBUF, D_MODEL, CCHUNK), jnp.float32),
            pltpu.SemaphoreType.DMA,
            pltpu.SemaphoreType.DMA,
            pltpu.SemaphoreType.DMA,
            pltpu.SemaphoreType.DMA,
        ],
    )
    def emb_kernel(x_hbm, table_hbm, out_hbm, idx_v, gbuf, tbuf,
                   gs0, gs1, os0, os1):
        gsems = (gs0, gs1)
        osems = (os0, os1)
        wid = lax.axis_index("s") * nc + lax.axis_index("c")
        block0 = wid * bpw

        pltpu.sync_copy(x_hbm.at[wid], idx_v)

        def gather(t, b):
            pltpu.async_copy(table_hbm.at[idx_v.at[t]], gbuf.at[b], gsems[b])

        def gwait(t, b):
            pltpu.make_async_copy(
                table_hbm.at[idx_v.at[t]], gbuf.at[b], gsems[b]).wait()

        def out_start(t, b):
            bid = block0 + t
            j, it = bid // n_it, bid % n_it
            for dt in range(d_hi):
                pltpu.async_copy(
                    tbuf.at[b, pl.ds(dt * 8, 8), :],
                    out_hbm.at[j, dt, it, :, :], osems[b])

        def owait(t, b):
            bid = block0 + t
            j, it = bid // n_it, bid % n_it
            for dt in range(d_hi):
                pltpu.make_async_copy(
                    tbuf.at[b, pl.ds(dt * 8, 8), :],
                    out_hbm.at[j, dt, it, :, :], osems[b]).wait()

        iota = lax.iota(jnp.int32, LANES)

        def transform(b):
            for k in range(CCHUNK // LANES):

                rowv = iota + (LANES * k)

                @functools.partial(plsc.parallel_loop, 0, D_MODEL, unroll=1)
                def _body(f):
                    colv = jnp.broadcast_to(f, (LANES,))
                    v = plsc.load_gather(gbuf.at[b], [rowv, colv])
                    tbuf[b, f, pl.ds(LANES * k, LANES)] = v * SCALE

        for b in range(NBUF):
            gather(b, b)
        for b in range(NBUF):
            gwait(b, b)
            transform(b)
            out_start(b, b)
            gather(b + NBUF, b)

        def block_pair(i, carry):
            for b in range(NBUF):
                t = i * NBUF + b
                gwait(t, b)
                owait(t - NBUF, b)
                transform(b)
                out_start(t, b)

                @pl.when(t + NBUF < bpw)
                def _():
                    gather(t + NBUF, b)
            return carry

        lax.fori_loop(1, bpw // NBUF, block_pair, 0)

        for b in range(NBUF):
            owait(bpw - NBUF + b, b)

    return emb_kernel, nw, n_it


def kernel(x, table):
    n_i, n_j = x.shape
    emb, nw, n_it = _build(n_i, n_j)
    x_t = x.T.reshape(nw, -1, CCHUNK)
    out5 = emb(x_t, table)
    return jnp.transpose(out5, (2, 4, 0, 1, 3)).reshape(n_i, n_j, D_MODEL)


# speedup vs baseline: 3.6812x; 1.4445x over previous
import functools
import math

import jax
import jax.numpy as jnp
from jax import lax
from jax.experimental import pallas as pl
from jax.experimental.pallas import tpu as pltpu
from jax.experimental.pallas import tpu_sc as plsc

D_MODEL = 64
SCALE = math.sqrt(D_MODEL)
LANES = 16
CCHUNK = 128
NBUF = 2
ROW_W = 128


@functools.lru_cache(maxsize=None)
def _build_fmt(vocab):
    info = plsc.get_sparse_core_info()
    nc, ns = info.num_cores, info.num_subcores
    nw = nc * ns
    n_full = vocab // CCHUNK
    tail = vocab - n_full * CCHUNK
    base_tiles = n_full // nw
    extra = n_full - base_tiles * nw
    assert base_tiles % NBUF == 0

    mesh = plsc.VectorSubcoreMesh(core_axis_name="c", subcore_axis_name="s")

    @functools.partial(
        pl.kernel,
        mesh=mesh,
        compiler_params=pltpu.CompilerParams(needs_layout_passes=False),
        out_type=jax.ShapeDtypeStruct((vocab, ROW_W), jnp.float32),
        scratch_types=[
            pltpu.VMEM((NBUF, D_MODEL, CCHUNK), jnp.float32),
            pltpu.VMEM((NBUF, CCHUNK, ROW_W), jnp.float32),
            pltpu.SemaphoreType.DMA,
            pltpu.SemaphoreType.DMA,
            pltpu.SemaphoreType.DMA,
            pltpu.SemaphoreType.DMA,
        ],
    )
    def fmt_kernel(tt_hbm, tail2_hbm, stg_hbm, gvbuf, sbuf,
                   gs0, gs1, os0, os1):
        gsems = (gs0, gs1)
        osems = (os0, os1)
        wid = lax.axis_index("s") * nc + lax.axis_index("c")
        v0 = wid * base_tiles

        def gin(v, b):
            pltpu.async_copy(
                tt_hbm.at[:, pl.ds(v * CCHUNK, CCHUNK)], gvbuf.at[b],
                gsems[b])

        def gwait(v, b):
            pltpu.make_async_copy(
                tt_hbm.at[:, pl.ds(v * CCHUNK, CCHUNK)], gvbuf.at[b],
                gsems[b]).wait()

        def out_start(v, b):
            pltpu.async_copy(
                sbuf.at[b], stg_hbm.at[pl.ds(v * CCHUNK, CCHUNK), :],
                osems[b])

        def owait(v, b):
            pltpu.make_async_copy(
                sbuf.at[b], stg_hbm.at[pl.ds(v * CCHUNK, CCHUNK), :],
                osems[b]).wait()

        iota = lax.iota(jnp.int32, LANES)

        def transform(b):
            for m in range(D_MODEL // LANES):
                rowv = iota + (LANES * m)

                @functools.partial(plsc.parallel_loop, 0, CCHUNK)
                def _body(s):
                    colv = jnp.broadcast_to(s, (LANES,))
                    v = plsc.load_gather(gvbuf.at[b], [rowv, colv])
                    sbuf[b, s, pl.ds(LANES * m, LANES)] = v * SCALE

        for b in range(NBUF):
            gin(v0 + b, b)
        for b in range(NBUF):
            gwait(v0 + b, b)
            transform(b)
            out_start(v0 + b, b)
            gin(v0 + b + NBUF, b)

        def tile_pair(i, carry):
            for b in range(NBUF):
                v = v0 + i * NBUF + b
                gwait(v, b)
                owait(v - NBUF, b)
                transform(b)
                out_start(v, b)

                @pl.when(i * NBUF + b + NBUF < base_tiles)
                def _():
                    gin(v + NBUF, b)
            return carry

        lax.fori_loop(1, base_tiles // NBUF, tile_pair, 0)
        for b in range(NBUF):
            owait(v0 + base_tiles - NBUF + b, b)

        if extra:
            @pl.when(wid < extra)
            def _():
                v = nw * base_tiles + wid
                pltpu.sync_copy(
                    tt_hbm.at[:, pl.ds(v * CCHUNK, CCHUNK)], gvbuf.at[0])
                transform(0)
                pltpu.sync_copy(
                    sbuf.at[0], stg_hbm.at[pl.ds(v * CCHUNK, CCHUNK), :])

        if tail:
            @pl.when(wid == extra)
            def _():
                pltpu.sync_copy(tail2_hbm, sbuf.at[0, pl.ds(0, tail), :])
                pltpu.sync_copy(
                    sbuf.at[0, pl.ds(0, tail), :],
                    stg_hbm.at[pl.ds(n_full * CCHUNK, tail), :])

    return fmt_kernel


@functools.lru_cache(maxsize=None)
def _build_emb(n_i, n_j, vocab):
    info = plsc.get_sparse_core_info()
    nc, ns = info.num_cores, info.num_subcores
    nw = nc * ns
    n_it = n_i // CCHUNK
    nblocks = n_j * n_it
    bpw = nblocks // nw
    assert n_it * CCHUNK == n_i and bpw * nw == nblocks
    d_hi = D_MODEL // 8

    mesh = plsc.VectorSubcoreMesh(core_axis_name="c", subcore_axis_name="s")

    @functools.partial(
        pl.kernel,
        mesh=mesh,
        compiler_params=pltpu.CompilerParams(needs_layout_passes=False),
        out_type=jax.ShapeDtypeStruct((n_j, d_hi, n_it, 8, CCHUNK),
                                      jnp.float32),
        scratch_types=[
            pltpu.VMEM((bpw, CCHUNK), jnp.int32),
            pltpu.VMEM((NBUF, CCHUNK, ROW_W), jnp.float32),
            pltpu.VMEM((NBUF, D_MODEL, CCHUNK), jnp.float32),
            pltpu.SemaphoreType.DMA,
            pltpu.SemaphoreType.DMA,
            pltpu.SemaphoreType.DMA,
            pltpu.SemaphoreType.DMA,
        ],
    )
    def emb_kernel(x_hbm, table_hbm, out_hbm, idx_v, gbuf, tbuf,
                   gs0, gs1, os0, os1):
        gsems = (gs0, gs1)
        osems = (os0, os1)
        wid = lax.axis_index("s") * nc + lax.axis_index("c")
        block0 = wid * bpw

        pltpu.sync_copy(x_hbm.at[wid], idx_v)

        def gather(t, b):
            pltpu.async_copy(table_hbm.at[idx_v.at[t]], gbuf.at[b],
                             gsems[b])

        def gwait(t, b):
            pltpu.make_async_copy(
                table_hbm.at[idx_v.at[t]], gbuf.at[b], gsems[b]).wait()

        def out_start(t, b):
            bid = block0 + t
            j, it = bid // n_it, bid % n_it
            for dt in range(d_hi):
                pltpu.async_copy(
                    tbuf.at[b, pl.ds(dt * 8, 8), :],
                    out_hbm.at[j, dt, it, :, :], osems[b])

        def owait(t, b):
            bid = block0 + t
            j, it = bid // n_it, bid % n_it
            for dt in range(d_hi):
                pltpu.make_async_copy(
                    tbuf.at[b, pl.ds(dt * 8, 8), :],
                    out_hbm.at[j, dt, it, :, :], osems[b]).wait()

        iota = lax.iota(jnp.int32, LANES)

        def transform(b):
            for k in range(CCHUNK // LANES):
                rowv = iota + (LANES * k)

                @functools.partial(plsc.parallel_loop, 0, D_MODEL)
                def _body(f):
                    colv = jnp.broadcast_to(f, (LANES,))
                    v = plsc.load_gather(gbuf.at[b], [rowv, colv])
                    tbuf[b, f, pl.ds(LANES * k, LANES)] = v

        for b in range(NBUF):
            gather(b, b)
        for b in range(NBUF):
            gwait(b, b)
            transform(b)
            out_start(b, b)
            gather(b + NBUF, b)

        def block_pair(i, carry):
            for b in range(NBUF):
                t = i * NBUF + b
                gwait(t, b)
                owait(t - NBUF, b)
                transform(b)
                out_start(t, b)

                @pl.when(t + NBUF < bpw)
                def _():
                    gather(t + NBUF, b)
            return carry

        lax.fori_loop(1, bpw // NBUF, block_pair, 0)

        for b in range(NBUF):
            owait(bpw - NBUF + b, b)

    return emb_kernel, nw, n_it


def kernel(x, table):
    n_i, n_j = x.shape
    vocab = table.shape[0]
    fmt = _build_fmt(vocab)
    emb, nw, n_it = _build_emb(n_i, n_j, vocab)
    tail = vocab % CCHUNK
    tail2 = jnp.pad(table[vocab - tail:] * SCALE,
                    ((0, 0), (0, ROW_W - D_MODEL)))
    staging = fmt(table.T, tail2)
    x_t = x.T.reshape(nw, -1, CCHUNK)
    out5 = emb(x_t, staging)
    return jnp.transpose(out5, (2, 4, 0, 1, 3)).reshape(n_i, n_j, D_MODEL)
